# baseline (speedup 1.0000x reference)
import functools

import jax
import jax.numpy as jnp
from jax import lax
from jax.experimental import pallas as pl
from jax.experimental.pallas import tpu as pltpu
from jax.experimental.pallas import tpu_sc as plsc

VOCAB = 100000
EMB = 128
BATCH = 4096

_NC = 2
_NS = 16
_L = 16
_NW = _NC * _NS
_BPW = BATCH // _NW

_A0, _C0 = 2 * 0 + 3, (1 + 1) * 1000003
_A1, _C1 = 2 * 2 + 3, (3 + 1) * 1000003


def _hash_mod(h):
    q = (h * jnp.float32(1e-5)).astype(jnp.int32).astype(jnp.float32)
    r = h - q * jnp.float32(100000.0)
    r = jnp.where(r < 0.0, r + jnp.float32(100000.0), r)
    r = jnp.where(r >= 100000.0, r - jnp.float32(100000.0), r)
    return r.astype(jnp.int32)


def _sc_body(x_hbm, w0_hbm, w1_hbm, g0_hbm, g1_hbm,
             x_v, idx0_v, idx1_v, rows0_v, rows1_v, gsem, osem):
    wid = lax.axis_index("s") * _NC + lax.axis_index("c")
    base = wid * _BPW
    pltpu.sync_copy(x_hbm.at[pl.ds(base, _BPW)], x_v)

    for i in range(0, _BPW, _L):
        sl = pl.ds(i, _L)
        xf = x_v[sl].astype(jnp.float32)
        idx0_v[sl] = _hash_mod(xf * jnp.float32(_A0) + jnp.float32(_C0))
        idx1_v[sl] = _hash_mod(xf * jnp.float32(_A1) + jnp.float32(_C1))

    rs = pl.ds(0, _BPW)
    c0 = pltpu.async_copy(w0_hbm.at[idx0_v.at[rs]], rows0_v, gsem)
    c1 = pltpu.async_copy(w1_hbm.at[idx1_v.at[rs]], rows1_v, gsem)
    ob = pl.ds(base, _BPW)
    c0.wait()
    o0 = pltpu.async_copy(rows0_v, g0_hbm.at[ob], osem)
    c1.wait()
    o1 = pltpu.async_copy(rows1_v, g1_hbm.at[ob], osem)
    o0.wait()
    o1.wait()


def _tc_combine(lw_ref, g0_ref, g1_ref, o_ref):
    w0 = lw_ref[0]
    w1 = lw_ref[1]
    rinv = 1.0 / (w0 + w1)
    o_ref[...] = g0_ref[...] * (w0 * rinv) + g1_ref[...] * (w1 * rinv)


_TC_BLK = 1024


@jax.jit
def kernel(x, W0, W1, lookup_weights):
    x = x.astype(jnp.int32)
    lw = lookup_weights.astype(jnp.float32).reshape(2)
    mesh = plsc.VectorSubcoreMesh(core_axis_name="c", subcore_axis_name="s")
    gather = functools.partial(
        pl.kernel,
        out_type=[
            jax.ShapeDtypeStruct((BATCH, EMB), jnp.float32),
            jax.ShapeDtypeStruct((BATCH, EMB), jnp.float32),
        ],
        mesh=mesh,
        scratch_types=[
            pltpu.VMEM((_BPW,), jnp.int32),
            pltpu.VMEM((_BPW,), jnp.int32),
            pltpu.VMEM((_BPW,), jnp.int32),
            pltpu.VMEM((_BPW, EMB), jnp.float32),
            pltpu.VMEM((_BPW, EMB), jnp.float32),
            pltpu.SemaphoreType.DMA,
            pltpu.SemaphoreType.DMA,
        ],
    )(_sc_body)
    g0, g1 = gather(x, W0, W1)
    return pl.pallas_call(
        _tc_combine,
        out_shape=jax.ShapeDtypeStruct((BATCH, EMB), jnp.float32),
        grid=(BATCH // _TC_BLK,),
        in_specs=[
            pl.BlockSpec(memory_space=pltpu.SMEM),
            pl.BlockSpec((_TC_BLK, EMB), lambda i: (i, 0)),
            pl.BlockSpec((_TC_BLK, EMB), lambda i: (i, 0)),
        ],
        out_specs=pl.BlockSpec((_TC_BLK, EMB), lambda i: (i, 0)),
    )(lw, g0, g1)


# speedup vs baseline: 1.1184x; 1.1184x over previous
import functools

import jax
import jax.numpy as jnp
from jax import lax
from jax.experimental import pallas as pl
from jax.experimental.pallas import tpu as pltpu
from jax.experimental.pallas import tpu_sc as plsc

VOCAB = 100000
EMB = 128
BATCH = 4096

_NC = 2
_NS = 16
_L = 16
_NW = _NC * _NS
_BPW = BATCH // _NW
_NG = 1
_GR = _BPW // _NG
_NW2 = 2

_A0, _C0 = 2 * 0 + 3, (1 + 1) * 1000003
_A1, _C1 = 2 * 2 + 3, (3 + 1) * 1000003


def _hash_mod(h):
    q = (h * jnp.float32(1e-5)).astype(jnp.int32).astype(jnp.float32)
    r = h - q * jnp.float32(100000.0)
    r = jnp.where(r < 0.0, r + jnp.float32(100000.0), r)
    r = jnp.where(r >= 100000.0, r - jnp.float32(100000.0), r)
    return r.astype(jnp.int32)


def _body(x_hbm, w0_hbm, w1_hbm, lw_hbm, out_hbm,
          x_v, idx0_v, idx1_v, rows0_v, rows1_v, w_v,
          gsems, osem):
    wid = lax.axis_index("s") * _NC + lax.axis_index("c")
    base = wid * _BPW
    pltpu.sync_copy(x_hbm.at[pl.ds(base, _BPW)], x_v)
    pltpu.sync_copy(lw_hbm, w_v.at[pl.ds(0, 2)])

    gathers = []
    for g in range(_NG):
        for i in range(g * _GR, (g + 1) * _GR, _L):
            sl = pl.ds(i, _L)
            xf = x_v[sl].astype(jnp.float32)
            idx0_v[sl] = _hash_mod(xf * jnp.float32(_A0) + jnp.float32(_C0))
            idx1_v[sl] = _hash_mod(xf * jnp.float32(_A1) + jnp.float32(_C1))
        rs = pl.ds(g * _GR, _GR)
        gathers.append((
            pltpu.async_copy(w0_hbm.at[idx0_v.at[rs]], rows0_v.at[rs], gsems[g]),
            pltpu.async_copy(w1_hbm.at[idx1_v.at[rs]], rows1_v.at[rs], gsems[g]),
        ))

    wv = w_v[...]
    w0v = jnp.broadcast_to(wv[0], (_L,))
    w1v = jnp.broadcast_to(wv[1], (_L,))
    rv = 1.0 / (w0v + w1v)
    s0 = w0v * rv
    s1 = w1v * rv

    for g in range(_NG):
        gathers[g][0].wait()
        gathers[g][1].wait()

    outs = []
    ck = _BPW // _NW2
    for c in range(_NW2):
        @plsc.parallel_loop(c * ck, (c + 1) * ck, unroll=4)
        def _(r):
            for j in range(EMB // _L):
                sl = pl.ds(j * _L, _L)
                rows0_v[r, sl] = rows0_v[r, sl] * s0 + rows1_v[r, sl] * s1

        rs = pl.ds(c * ck, ck)
        outs.append(pltpu.async_copy(
            rows0_v.at[rs], out_hbm.at[pl.ds(base + c * ck, ck)], osem))
    for cp in outs:
        cp.wait()


@jax.jit
def kernel(x, W0, W1, lookup_weights):
    x = x.astype(jnp.int32)
    lw = lookup_weights.astype(jnp.float32).reshape(2)
    mesh = plsc.VectorSubcoreMesh(core_axis_name="c", subcore_axis_name="s")
    run = functools.partial(
        pl.kernel,
        out_type=jax.ShapeDtypeStruct((BATCH, EMB), jnp.float32),
        mesh=mesh,
        scratch_types=[
            pltpu.VMEM((_BPW,), jnp.int32),
            pltpu.VMEM((_BPW,), jnp.int32),
            pltpu.VMEM((_BPW,), jnp.int32),
            pltpu.VMEM((_BPW, EMB), jnp.float32),
            pltpu.VMEM((_BPW, EMB), jnp.float32),
            pltpu.VMEM((_L,), jnp.float32),
            [pltpu.SemaphoreType.DMA] * _NG,
            pltpu.SemaphoreType.DMA,
        ],
    )(_body)
    return run(x, W0, W1, lw)


# speedup vs baseline: 1.1491x; 1.0274x over previous
import functools

import jax
import jax.numpy as jnp
from jax import lax
from jax.experimental import pallas as pl
from jax.experimental.pallas import tpu as pltpu
from jax.experimental.pallas import tpu_sc as plsc

VOCAB = 100000
EMB = 128
BATCH = 4096

_NC = 2
_NS = 16
_L = 16
_NW = _NC * _NS
_BPW = BATCH // _NW
_NG = 1
_GR = _BPW // _NG

_A0, _C0 = 2 * 0 + 3, (1 + 1) * 1000003
_A1, _C1 = 2 * 2 + 3, (3 + 1) * 1000003


def _hash_mod(h):
    q = (h * jnp.float32(1e-5)).astype(jnp.int32).astype(jnp.float32)
    r = h - q * jnp.float32(100000.0)
    r = jnp.where(r < 0.0, r + jnp.float32(100000.0), r)
    r = jnp.where(r >= 100000.0, r - jnp.float32(100000.0), r)
    return r.astype(jnp.int32)


def _body(x_hbm, w0_hbm, w1_hbm, lw_hbm, out_hbm,
          x_v, idx0_v, idx1_v, rows0_v, rows1_v, w_v,
          gsems, osem):
    wid = lax.axis_index("s") * _NC + lax.axis_index("c")
    base = wid * _BPW
    pltpu.sync_copy(x_hbm.at[pl.ds(base, _BPW)], x_v)

    gathers = []
    for g in range(_NG):
        for i in range(g * _GR, (g + 1) * _GR, _L):
            sl = pl.ds(i, _L)
            xf = x_v[sl].astype(jnp.float32)
            idx0_v[sl] = _hash_mod(xf * jnp.float32(_A0) + jnp.float32(_C0))
            idx1_v[sl] = _hash_mod(xf * jnp.float32(_A1) + jnp.float32(_C1))
        rs = pl.ds(g * _GR, _GR)
        gathers.append((
            pltpu.async_copy(w0_hbm.at[idx0_v.at[rs]], rows0_v.at[rs], gsems[g]),
            pltpu.async_copy(w1_hbm.at[idx1_v.at[rs]], rows1_v.at[rs], gsems[g]),
        ))

    pltpu.sync_copy(lw_hbm, w_v.at[pl.ds(0, 2)])
    wv = w_v[...]
    w0v = jnp.broadcast_to(wv[0], (_L,))
    w1v = jnp.broadcast_to(wv[1], (_L,))
    rv = 1.0 / (w0v + w1v)
    s0 = w0v * rv
    s1 = w1v * rv

    outs = []
    for g in range(_NG):
        gathers[g][0].wait()
        gathers[g][1].wait()

        @plsc.parallel_loop(g * _GR, (g + 1) * _GR, unroll=4)
        def _(r):
            for j in range(EMB // _L):
                sl = pl.ds(j * _L, _L)
                rows0_v[r, sl] = rows0_v[r, sl] * s0 + rows1_v[r, sl] * s1

        rs = pl.ds(g * _GR, _GR)
        outs.append(pltpu.async_copy(
            rows0_v.at[rs], out_hbm.at[pl.ds(base + g * _GR, _GR)], osem))
    for cp in outs:
        cp.wait()


@jax.jit
def kernel(x, W0, W1, lookup_weights):
    x = x.astype(jnp.int32)
    lw = lookup_weights.astype(jnp.float32).reshape(2)
    mesh = plsc.VectorSubcoreMesh(core_axis_name="c", subcore_axis_name="s")
    run = functools.partial(
        pl.kernel,
        out_type=jax.ShapeDtypeStruct((BATCH, EMB), jnp.float32),
        mesh=mesh,
        scratch_types=[
            pltpu.VMEM((_BPW,), jnp.int32),
            pltpu.VMEM((_BPW,), jnp.int32),
            pltpu.VMEM((_BPW,), jnp.int32),
            pltpu.VMEM((_BPW, EMB), jnp.float32),
            pltpu.VMEM((_BPW, EMB), jnp.float32),
            pltpu.VMEM((_L,), jnp.float32),
            [pltpu.SemaphoreType.DMA] * _NG,
            pltpu.SemaphoreType.DMA,
        ],
    )(_body)
    return run(x, W0, W1, lw)
